# baseline (speedup 1.0000x reference)
import functools

import jax
import jax.numpy as jnp
from jax import lax
from jax.experimental import pallas as pl
from jax.experimental.pallas import tpu as pltpu
from jax.experimental.pallas import tpu_sc as plsc

NUM_EXPERTS = 16
HIDDEN = 1024
FF = 2048
TOKENS = 2048
CAP = 256
NSLOTS = NUM_EXPERTS * CAP

_CHUNK = 512
_NW = 32
_BPW = TOKENS // _NW

def _scalar_mesh():
    return plsc.ScalarSubcoreMesh(axis_name="core", num_cores=2)


def _vector_mesh():
    return plsc.VectorSubcoreMesh(core_axis_name="c", subcore_axis_name="s")


def _route(idx):
    @functools.partial(
        pl.kernel,
        out_type=[
            jax.ShapeDtypeStruct((TOKENS,), jnp.int32),
            jax.ShapeDtypeStruct((NUM_EXPERTS,), jnp.int32),
        ],
        mesh=_scalar_mesh(),
        scratch_types=[
            pltpu.SMEM((_CHUNK,), jnp.int32),
            pltpu.SMEM((_CHUNK,), jnp.int32),
            pltpu.SMEM((NUM_EXPERTS,), jnp.int32),
            pltpu.SMEM((2,), jnp.int32),
            pltpu.SemaphoreType.DMA,
        ],
    )
    def k(idx_hbm, slot_hbm, cnt_hbm, idx_s, slot_s, cnt_s, misc_s, sem):
        @pl.when(lax.axis_index("core") == 0)
        def _():
            @pl.loop(0, NUM_EXPERTS)
            def _(e):
                cnt_s[e] = 0

            misc_s[0] = 0

            @pl.loop(0, TOKENS // _CHUNK)
            def _(c):
                pltpu.async_copy(
                    idx_hbm.at[pl.ds(c * _CHUNK, _CHUNK)], idx_s, sem
                ).wait()

                @pl.loop(0, _CHUNK)
                def _(t):
                    e = idx_s[t]
                    r = cnt_s[e]
                    cnt_s[e] = r + 1
                    ok = r < CAP
                    slot_s[t] = jnp.where(ok, e * CAP + r, -1)
                    misc_s[0] = jnp.where(ok, misc_s[0], 1)

                pltpu.async_copy(
                    slot_s, slot_hbm.at[pl.ds(c * _CHUNK, _CHUNK)], sem
                ).wait()

            @pl.loop(0, NUM_EXPERTS)
            def _(i):
                e = NUM_EXPERTS - 1 - i

                @pl.when(cnt_s[e] < CAP)
                def _():
                    misc_s[1] = e * CAP + cnt_s[e]

            @pl.when(misc_s[0] > 0)
            def _():
                @pl.loop(0, TOKENS // _CHUNK)
                def _(c):
                    pltpu.async_copy(
                        slot_hbm.at[pl.ds(c * _CHUNK, _CHUNK)], slot_s, sem
                    ).wait()

                    @pl.loop(0, _CHUNK)
                    def _(t):
                        s = slot_s[t]
                        slot_s[t] = jnp.where(s < 0, misc_s[1], s)

                    pltpu.async_copy(
                        slot_s, slot_hbm.at[pl.ds(c * _CHUNK, _CHUNK)], sem
                    ).wait()

            pltpu.async_copy(cnt_s, cnt_hbm, sem).wait()

    return k(idx)


def _dispatch(hidden, slots):
    @functools.partial(
        pl.kernel,
        out_type=jax.ShapeDtypeStruct((NSLOTS, HIDDEN), jnp.float32),
        mesh=_vector_mesh(),
        scratch_types=[
            pltpu.VMEM((1, _BPW), jnp.int32),
            pltpu.VMEM((_BPW, HIDDEN), jnp.float32),
            pltpu.SemaphoreType.DMA,
            pltpu.SemaphoreType.DMA,
        ],
    )
    def k(hid_hbm, slot_hbm, xg_hbm, idx_v, rows_v, sem1, sem2):
        wid = lax.axis_index("s") * 2 + lax.axis_index("c")
        base = wid * _BPW
        c1 = pltpu.async_copy(slot_hbm.at[pl.ds(base, _BPW)], idx_v.at[0], sem1)
        c2 = pltpu.async_copy(hid_hbm.at[pl.ds(base, _BPW)], rows_v, sem2)
        c1.wait()
        c2.wait()
        pltpu.sync_copy(rows_v, xg_hbm.at[idx_v.at[0]])

    return k(hidden, slots)


def _collect(y, slots):
    @functools.partial(
        pl.kernel,
        out_type=jax.ShapeDtypeStruct((TOKENS, HIDDEN), jnp.float32),
        mesh=_vector_mesh(),
        scratch_types=[
            pltpu.VMEM((_BPW,), jnp.int32),
            pltpu.VMEM((_BPW, HIDDEN), jnp.float32),
            pltpu.SemaphoreType.DMA,
        ],
    )
    def k(y_hbm, slot_hbm, out_hbm, idx_v, rows_v, sem):
        wid = lax.axis_index("s") * 2 + lax.axis_index("c")
        base = wid * _BPW
        pltpu.sync_copy(slot_hbm.at[pl.ds(base, _BPW)], idx_v)
        pltpu.async_copy(y_hbm.at[idx_v], rows_v, sem).wait()
        pltpu.sync_copy(rows_v, out_hbm.at[pl.ds(base, _BPW)])

    return k(y, slots)


_FB = 512


def _mlp_body(cnt_ref, xg_ref, g_ref, u_ref, d_ref, y_ref):
    e = pl.program_id(0)
    f = pl.program_id(1)
    cnt = cnt_ref[e]
    rows = lax.broadcasted_iota(jnp.int32, (CAP, 1), 0)
    x = jnp.where(rows < cnt, xg_ref[...], 0.0)
    gate = jnp.dot(x, g_ref[0], preferred_element_type=jnp.float32)
    up = jnp.dot(x, u_ref[0], preferred_element_type=jnp.float32)
    h = up * (gate * jax.nn.sigmoid(gate))
    part = jnp.dot(h, d_ref[0], preferred_element_type=jnp.float32)

    @pl.when(f == 0)
    def _():
        y_ref[...] = part

    @pl.when(f != 0)
    def _():
        y_ref[...] += part


def _expert_mlp(xg, counts, gate_up_proj, down_proj):
    grid = (NUM_EXPERTS, FF // _FB)
    return pl.pallas_call(
        _mlp_body,
        grid_spec=pltpu.PrefetchScalarGridSpec(
            num_scalar_prefetch=1,
            grid=grid,
            in_specs=[
                pl.BlockSpec((CAP, HIDDEN), lambda e, f, cnt: (e, 0)),
                pl.BlockSpec((1, HIDDEN, _FB), lambda e, f, cnt: (e, 0, f)),
                pl.BlockSpec((1, HIDDEN, _FB), lambda e, f, cnt: (e, 0, f + FF // _FB)),
                pl.BlockSpec((1, _FB, HIDDEN), lambda e, f, cnt: (e, f, 0)),
            ],
            out_specs=pl.BlockSpec((CAP, HIDDEN), lambda e, f, cnt: (e, 0)),
        ),
        out_shape=jax.ShapeDtypeStruct((NSLOTS, HIDDEN), jnp.float32),
        compiler_params=pltpu.CompilerParams(
            dimension_semantics=("arbitrary", "arbitrary"),
        ),
    )(counts, xg, gate_up_proj, gate_up_proj, down_proj)


def kernel(hidden_states, local_expert_indices, gate_up_proj, down_proj):
    idx = local_expert_indices.astype(jnp.int32)
    slots, counts = _route(idx)
    xg = _dispatch(hidden_states, slots)
    y = _expert_mlp(xg, counts, gate_up_proj, down_proj)
    return _collect(y, slots)


# speedup vs baseline: 1.2349x; 1.2349x over previous
import dataclasses
import functools

import jax
import jax.numpy as jnp
from jax import lax
from jax.experimental import pallas as pl
from jax.experimental.pallas import tpu as pltpu
from jax.experimental.pallas import tpu_sc as plsc

NUM_EXPERTS = 16
HIDDEN = 1024
FF = 2048
TOKENS = 2048
CAP = 256
NSLOTS = NUM_EXPERTS * CAP

_CHUNK = 512
_NW = 32
_BPW = TOKENS // _NW

def _scalar_mesh():
    return plsc.ScalarSubcoreMesh(axis_name="core", num_cores=2)


def _vector_mesh():
    return plsc.VectorSubcoreMesh(core_axis_name="c", subcore_axis_name="s")


_NL = 16
_TPS = TOKENS // _NL
_VPS = _TPS // _NL


def _sc_compiler_params():
    cp = pltpu.CompilerParams()
    if "needs_layout_passes" in pltpu.CompilerParams.__dataclass_fields__:
        cp = dataclasses.replace(cp, needs_layout_passes=False)
    return cp


def _route_dispatch(hidden, idx):
    @functools.partial(
        pl.kernel,
        out_type=[
            jax.ShapeDtypeStruct((NSLOTS, HIDDEN), jnp.float32),
            jax.ShapeDtypeStruct((TOKENS,), jnp.int32),
            jax.ShapeDtypeStruct((NUM_EXPERTS,), jnp.int32),
        ],
        mesh=_vector_mesh(),
        scratch_types=[
            pltpu.VMEM((_BPW, HIDDEN), jnp.float32),
            pltpu.VMEM((_TPS,), jnp.int32),
            pltpu.VMEM((_TPS,), jnp.int32),
            pltpu.VMEM((_NL,), jnp.int32),
            pltpu.VMEM((_NL * _NL,), jnp.int32),
            pltpu.VMEM((1, _BPW), jnp.int32),
            pltpu.VMEM((_NL,), jnp.int32),
            pltpu.VMEM_SHARED((_NL * _NL,), jnp.int32),
            pltpu.SemaphoreType.DMA,
        ],
        compiler_params=_sc_compiler_params(),
    )
    def k(hid_hbm, idx_hbm, xg_hbm, slot_hbm, cnt_hbm,
          rows_v, idx_v, slot_v, h_v, hmat_v, sidx_v, cout_v, sh_h, sem):
        c = lax.axis_index("c")
        s = lax.axis_index("s")
        wid = s * 2 + c
        rows_dma = pltpu.async_copy(
            hid_hbm.at[pl.ds(wid * _BPW, _BPW)], rows_v, sem
        )
        pltpu.sync_copy(idx_hbm.at[pl.ds(s * _TPS, _TPS)], idx_v)

        lanes = lax.iota(jnp.int32, _NL)
        vregs = [idx_v[pl.ds(_NL * i, _NL)] for i in range(_VPS)]

        h = jnp.zeros((_NL,), jnp.int32)
        for i in range(_VPS):
            for e in range(NUM_EXPERTS):
                n_e = jnp.sum((vregs[i] == e).astype(jnp.int32))
                h = jnp.where(lanes == e, h + n_e, h)
        h_v[...] = h
        pltpu.sync_copy(h_v, sh_h.at[pl.ds(s * _NL, _NL)])
        plsc.subcore_barrier()
        pltpu.sync_copy(sh_h, hmat_v)

        off = jnp.zeros((_NL,), jnp.int32)
        tot = jnp.zeros((_NL,), jnp.int32)
        for sp in range(_NL):
            row = hmat_v[pl.ds(sp * _NL, _NL)]
            off = off + jnp.where(sp < s, row, 0)
            tot = tot + row
        off_e = [jnp.sum(jnp.where(lanes == e, off, 0)) for e in range(NUM_EXPERTS)]

        pad = jnp.min(jnp.where(tot < CAP, lanes * CAP + tot, NSLOTS * 2))

        for i in range(_VPS):
            v = vregs[i]
            rank = jnp.zeros((_NL,), jnp.int32)
            for e in range(NUM_EXPERTS):
                m = v == e
                ones = m.astype(jnp.int32)
                pre = jnp.cumsum(ones)
                rank = rank + jnp.where(m, off_e[e] + pre - 1, 0)
                off_e[e] = off_e[e] + jnp.sum(ones)
            slot = v * CAP + rank
            slot_v[pl.ds(_NL * i, _NL)] = jnp.where(rank < CAP, slot, pad)

        @pl.when(c == 0)
        def _():
            pltpu.sync_copy(slot_v, slot_hbm.at[pl.ds(s * _TPS, _TPS)])

            @pl.when(s == 0)
            def _():
                cout_v[...] = tot
                pltpu.sync_copy(cout_v, cnt_hbm)

        for j in range(_BPW // _NL):
            sidx_v[0, pl.ds(_NL * j, _NL)] = slot_v[
                pl.ds(c * _BPW + _NL * j, _NL)
            ]
        rows_dma.wait()
        pltpu.sync_copy(rows_v, xg_hbm.at[sidx_v.at[0]])

    return k(hidden, idx)


def _collect(y, slots):
    @functools.partial(
        pl.kernel,
        out_type=jax.ShapeDtypeStruct((TOKENS, HIDDEN), jnp.float32),
        mesh=_vector_mesh(),
        scratch_types=[
            pltpu.VMEM((_BPW,), jnp.int32),
            pltpu.VMEM((_BPW, HIDDEN), jnp.float32),
            pltpu.SemaphoreType.DMA,
        ],
    )
    def k(y_hbm, slot_hbm, out_hbm, idx_v, rows_v, sem):
        wid = lax.axis_index("s") * 2 + lax.axis_index("c")
        base = wid * _BPW
        pltpu.sync_copy(slot_hbm.at[pl.ds(base, _BPW)], idx_v)
        pltpu.async_copy(y_hbm.at[idx_v], rows_v, sem).wait()
        pltpu.sync_copy(rows_v, out_hbm.at[pl.ds(base, _BPW)])

    return k(y, slots)


_FB = 1024


def _mlp_body(cnt_ref, xg_ref, g_ref, u_ref, d_ref, y_ref):
    e = pl.program_id(0)
    f = pl.program_id(1)
    cnt = cnt_ref[e]
    rows = lax.broadcasted_iota(jnp.int32, (CAP, 1), 0)
    x = jnp.where(rows < cnt, xg_ref[...], 0.0)
    gate = jnp.dot(x, g_ref[0], preferred_element_type=jnp.float32)
    up = jnp.dot(x, u_ref[0], preferred_element_type=jnp.float32)
    h = up * (gate * jax.nn.sigmoid(gate))
    part = jnp.dot(h, d_ref[0], preferred_element_type=jnp.float32)

    @pl.when(f == 0)
    def _():
        y_ref[...] = part

    @pl.when(f != 0)
    def _():
        y_ref[...] += part


def _expert_mlp(xg, counts, gate_up_proj, down_proj):
    grid = (NUM_EXPERTS, FF // _FB)
    return pl.pallas_call(
        _mlp_body,
        grid_spec=pltpu.PrefetchScalarGridSpec(
            num_scalar_prefetch=1,
            grid=grid,
            in_specs=[
                pl.BlockSpec((CAP, HIDDEN), lambda e, f, cnt: (e, 0)),
                pl.BlockSpec((1, HIDDEN, _FB), lambda e, f, cnt: (e, 0, f)),
                pl.BlockSpec((1, HIDDEN, _FB), lambda e, f, cnt: (e, 0, f + FF // _FB)),
                pl.BlockSpec((1, _FB, HIDDEN), lambda e, f, cnt: (e, f, 0)),
            ],
            out_specs=pl.BlockSpec((CAP, HIDDEN), lambda e, f, cnt: (e, 0)),
        ),
        out_shape=jax.ShapeDtypeStruct((NSLOTS, HIDDEN), jnp.float32),
        compiler_params=pltpu.CompilerParams(
            dimension_semantics=("arbitrary", "arbitrary"),
        ),
    )(counts, xg, gate_up_proj, gate_up_proj, down_proj)


def kernel(hidden_states, local_expert_indices, gate_up_proj, down_proj):
    idx = local_expert_indices.astype(jnp.int32)
    xg, slots, counts = _route_dispatch(hidden_states, idx)
    y = _expert_mlp(xg, counts, gate_up_proj, down_proj)
    return _collect(y, slots)
